# baseline (speedup 1.0000x reference)
import functools

import jax
import jax.numpy as jnp
from jax import lax
from jax.experimental import pallas as pl
from jax.experimental.pallas import tpu as pltpu
from jax.experimental.pallas import tpu_sc as plsc

_N = 10000
_E = 320000
_D = 128
_DH = _D // 2
_NP = 10240
_NC = 2
_NS = 16
_NW = _NC * _NS
_B = 128
_EP = 327680
_NB = _EP // _NW // _B
_NB2 = _EP // _NS // _B
_TS = _NP // _NS

_mesh = plsc.VectorSubcoreMesh(core_axis_name="c", subcore_axis_name="s")


def _fill_1d(ref, n, value):
    def body(i, carry):
        ref[pl.ds(i * 16, 16)] = jnp.full((16,), value, jnp.float32)
        return carry
    lax.fori_loop(0, n // 16, body, 0)


@functools.partial(
    pl.kernel,
    out_type=jax.ShapeDtypeStruct((_NC, 2, _NP), jnp.float32),
    mesh=_mesh,
    compiler_params=pltpu.CompilerParams(use_tc_tiling_on_sc=False),
    scratch_types=[
        pltpu.VMEM((_NB * _B,), jnp.int32),
        pltpu.VMEM((_NB * _B,), jnp.int32),
        pltpu.VMEM((128,), jnp.float32),
        pltpu.VMEM((_TS,), jnp.float32),
        pltpu.VMEM_SHARED((_NP,), jnp.float32),
        pltpu.VMEM_SHARED((_NP,), jnp.float32),
        pltpu.SemaphoreType.DMA,
    ],
)
def _deg_kernel(src_hbm, dst_hbm, deg_hbm, src_v, dst_v, ones_v, zero_v,
                dego_sh, degi_sh, sem):
    c = lax.axis_index("c")
    s = lax.axis_index("s")
    half = pl.ds(c * (_NB * _B), _NB * _B)
    pltpu.sync_copy(src_hbm.at[s, half], src_v)
    pltpu.sync_copy(dst_hbm.at[s, half], dst_v)
    _fill_1d(ones_v, 128, 1.0)
    _fill_1d(zero_v, _TS, 0.0)
    pltpu.sync_copy(zero_v, dego_sh.at[pl.ds(s * _TS, _TS)])
    pltpu.sync_copy(zero_v, degi_sh.at[pl.ds(s * _TS, _TS)])
    plsc.subcore_barrier()

    def body(q, carry):
        for u in range(4):
            j = q * 4 + u
            pltpu.async_copy(ones_v.at[pl.ds(0, _B)],
                             dego_sh.at[src_v.at[pl.ds(j * _B, _B)]],
                             sem, add=True)
            pltpu.async_copy(ones_v.at[pl.ds(0, _B)],
                             degi_sh.at[dst_v.at[pl.ds(j * _B, _B)]],
                             sem, add=True)
        for u in range(8):
            pltpu.make_async_copy(ones_v.at[pl.ds(0, _B)],
                                  dego_sh.at[src_v.at[pl.ds(0, _B)]],
                                  sem).wait()
        return carry
    lax.fori_loop(0, _NB // 4, body, 0)
    plsc.subcore_barrier()
    sl = pl.ds(s * _TS, _TS)
    pltpu.sync_copy(dego_sh.at[sl], deg_hbm.at[c, 0, sl])
    pltpu.sync_copy(degi_sh.at[sl], deg_hbm.at[c, 1, sl])


def _prep_body(x_ref, deg_ref, xs_ref, norm_ref, nd1_ref):
    deg = deg_ref[0] + deg_ref[1]
    norm = lax.rsqrt(jnp.maximum(deg, 1.0))
    norm_ref[...] = norm
    row = lax.broadcasted_iota(jnp.int32, (1, _NP), 1)
    nd1_ref[...] = jnp.reshape(
        jnp.where(row < _N, norm[1:2, :], 0.0), (_NP,))
    ns_col = jnp.reshape(norm[0, :_N], (_N, 1))
    x = x_ref[...]
    mask = (jnp.sum(x, axis=1, keepdims=True) != 0.0).astype(jnp.float32)
    xs_ref[...] = x * (mask * ns_col)


_RING = 4


@functools.partial(
    pl.kernel,
    out_type=(
        jax.ShapeDtypeStruct((_NP, _D), jnp.float32),
        jax.ShapeDtypeStruct((_NC, _NP), jnp.float32),
    ),
    mesh=_mesh,
    compiler_params=pltpu.CompilerParams(use_tc_tiling_on_sc=False),
    scratch_types=[
        pltpu.VMEM((_NB2 * _B,), jnp.int32),
        pltpu.VMEM((_NB * _B,), jnp.int32),
        pltpu.VMEM((_NB2 * _B,), jnp.int32),
        pltpu.VMEM((_B, _DH), jnp.float32),
        pltpu.VMEM((_B, _DH), jnp.float32),
        pltpu.VMEM((_B, _DH), jnp.float32),
        pltpu.VMEM((_B, _DH), jnp.float32),
        pltpu.VMEM((_B,), jnp.float32),
        pltpu.VMEM((_B,), jnp.float32),
        pltpu.VMEM((64, _DH), jnp.float32),
        pltpu.VMEM((_TS,), jnp.float32),
        pltpu.VMEM_SHARED((_NP, _DH), jnp.float32),
        pltpu.VMEM_SHARED((_NP,), jnp.float32),
        pltpu.SemaphoreType.DMA,
        pltpu.SemaphoreType.DMA,
        pltpu.SemaphoreType.DMA,
        pltpu.SemaphoreType.DMA,
        pltpu.SemaphoreType.DMA,
        pltpu.SemaphoreType.DMA,
    ],
)
def _edge_kernel(xsr_hbm, nd_hbm, src_hbm, dst_hbm, agg_hbm, t_hbm,
                 srcx_v, srcp_v, dst_v, rows0, rows1, rows2, rows3,
                 nv0, nv1, zero_v, zt_v, agg_sh, t_sh,
                 semr0, semr1, semr2, semr3, semn0, semn1):
    c = lax.axis_index("c")
    s = lax.axis_index("s")
    rows = (rows0, rows1, rows2, rows3)
    semr = (semr0, semr1, semr2, semr3)
    nvals = (nv0, nv1)
    semn = (semn0, semn1)
    pltpu.sync_copy(src_hbm.at[s], srcx_v)
    pltpu.sync_copy(src_hbm.at[s, pl.ds(c * (_NB * _B), _NB * _B)], srcp_v)
    pltpu.sync_copy(dst_hbm.at[s], dst_v)

    def dbl(i, carry):
        for k in range(10):
            sl = pl.ds((i * 10 + k) * 16, 16)
            v = srcx_v[sl]
            srcx_v[sl] = v + v + c
        return carry
    lax.fori_loop(0, _NB2 * _B // 160, dbl, 0)

    for u in range(_RING):
        pltpu.async_copy(xsr_hbm.at[srcx_v.at[pl.ds(u * _B, _B)]], rows[u],
                         semr[u])
    for v in range(2):
        pltpu.async_copy(nd_hbm.at[dst_v.at[pl.ds((c * _NB + v) * _B, _B)]],
                         nvals[v], semn[v])

    def zf(i, carry):
        for k in range(_DH // 16):
            zero_v[i, pl.ds(k * 16, 16)] = jnp.zeros((16,), jnp.float32)
        return carry
    lax.fori_loop(0, 64, zf, 0)
    _fill_1d(zt_v, _TS, 0.0)
    for q in range(_TS // 64):
        pltpu.sync_copy(zero_v, agg_sh.at[pl.ds(s * _TS + q * 64, 64), :])
    pltpu.sync_copy(zt_v, t_sh.at[pl.ds(s * _TS, _TS)])
    plsc.subcore_barrier()

    def body(q, carry):
        for u in range(_RING):
            j = q * _RING + u
            jp = jnp.minimum(j + _RING, _NB2 - 1)
            pltpu.make_async_copy(xsr_hbm.at[srcx_v.at[pl.ds(j * _B, _B)]],
                                  rows[u], semr[u]).wait()
            pltpu.sync_copy(rows[u], agg_sh.at[dst_v.at[pl.ds(j * _B, _B)]],
                            add=True)
            pltpu.async_copy(xsr_hbm.at[srcx_v.at[pl.ds(jp * _B, _B)]],
                             rows[u], semr[u])
            if u % 2 == 1:
                v = u // 2
                jj = q * 2 + v
                jjp = jnp.minimum(c * _NB + jj + 2, _NB2 - 1)
                pltpu.make_async_copy(
                    nd_hbm.at[dst_v.at[pl.ds((c * _NB + jj) * _B, _B)]],
                    nvals[v], semn[v]).wait()
                pltpu.sync_copy(nvals[v],
                                t_sh.at[srcp_v.at[pl.ds(jj * _B, _B)]],
                                add=True)
                pltpu.async_copy(nd_hbm.at[dst_v.at[pl.ds(jjp * _B, _B)]],
                                 nvals[v], semn[v])
        return carry
    lax.fori_loop(0, _NB2 // _RING, body, 0)

    for u in range(_RING):
        pltpu.make_async_copy(xsr_hbm.at[srcx_v.at[pl.ds(0, _B)]], rows[u],
                              semr[u]).wait()
    for v in range(2):
        pltpu.make_async_copy(nd_hbm.at[dst_v.at[pl.ds(0, _B)]], nvals[v],
                              semn[v]).wait()
    plsc.subcore_barrier()
    sl = pl.ds(s * _TS, _TS)
    pltpu.sync_copy(agg_sh.at[sl, :], agg_hbm.at[sl, pl.ds(c * _DH, _DH)])
    pltpu.sync_copy(t_sh.at[sl], t_hbm.at[c, sl])


_RB = 1280


def _final_body(agg_ref, t_ref, norm_ref, w0_ref, b0_ref, w1_ref,
                b1_ref, out_ref, acc_ref):
    i = pl.program_id(0)

    @pl.when(i == 0)
    def _init():
        acc_ref[...] = jnp.zeros_like(acc_ref)

    a = agg_ref[...]
    nd_col = jnp.reshape(norm_ref[1], (_RB, 1))
    z = jnp.dot(a * nd_col, w0_ref[...], preferred_element_type=jnp.float32)
    h = jnp.maximum(z + b0_ref[...], 0.0)
    cvec = norm_ref[0] * (t_ref[0] + t_ref[1])
    acc_ref[...] += jnp.sum(h * jnp.reshape(cvec, (_RB, 1)), axis=0,
                            keepdims=True)

    @pl.when(i == pl.num_programs(0) - 1)
    def _fin():
        v = acc_ref[...] * (1.0 / _N)
        out_ref[...] = (
            jnp.dot(v, w1_ref[...], preferred_element_type=jnp.float32)
            + b1_ref[...]
        )


def kernel(x, edge_index, W0, b0, W1, b1):
    src = edge_index[0]
    dst = edge_index[1]
    npad = _EP - _E
    trash = jnp.full((npad,), _NP - 1, src.dtype)
    srck = jnp.concatenate([src, trash]).reshape(_NS, _NB2 * _B)
    srce = jnp.concatenate([src, jnp.zeros((npad,), src.dtype)]).reshape(
        _NS, _NB2 * _B)
    dstf = jnp.concatenate([dst, trash]).reshape(_NS, _NB2 * _B)

    deg = _deg_kernel(srck, dstf)

    xs, norm, nd1 = pl.pallas_call(
        _prep_body,
        out_shape=(
            jax.ShapeDtypeStruct((_N, _D), jnp.float32),
            jax.ShapeDtypeStruct((2, _NP), jnp.float32),
            jax.ShapeDtypeStruct((_NP,), jnp.float32),
        ),
    )(x, deg)

    agg, t = _edge_kernel(xs.reshape(2 * _N, _DH), nd1, srce, dstf)

    out = pl.pallas_call(
        _final_body,
        grid=(_NP // _RB,),
        in_specs=[
            pl.BlockSpec((_RB, _D), lambda i: (i, 0)),
            pl.BlockSpec((_NC, _RB), lambda i: (0, i)),
            pl.BlockSpec((2, _RB), lambda i: (0, i)),
            pl.BlockSpec((_D, _D), lambda i: (0, 0)),
            pl.BlockSpec((1, _D), lambda i: (0, 0)),
            pl.BlockSpec((_D, _D), lambda i: (0, 0)),
            pl.BlockSpec((1, _D), lambda i: (0, 0)),
        ],
        out_specs=pl.BlockSpec((1, _D), lambda i: (0, 0)),
        out_shape=jax.ShapeDtypeStruct((1, _D), jnp.float32),
        scratch_shapes=[pltpu.VMEM((1, _D), jnp.float32)],
    )(agg, t, norm, W0, b0.reshape(1, _D), W1, b1.reshape(1, _D))
    return out


# speedup vs baseline: 2.7004x; 2.7004x over previous
import functools

import jax
import jax.numpy as jnp
from jax import lax
from jax.experimental import pallas as pl
from jax.experimental.pallas import tpu as pltpu
from jax.experimental.pallas import tpu_sc as plsc

_N = 10000
_E = 320000
_D = 128
_DH = _D // 2
_NP = 10240
_NC = 2
_NS = 16
_NW = _NC * _NS
_B = 125
_NB = _E // _NW // _B
_NB2 = _E // _NS // _B
_TS = _NP // _NS

_mesh = plsc.VectorSubcoreMesh(core_axis_name="c", subcore_axis_name="s")


def _fill_1d(ref, n, value):
    def body(i, carry):
        ref[pl.ds(i * 16, 16)] = jnp.full((16,), value, jnp.float32)
        return carry
    lax.fori_loop(0, n // 16, body, 0)


@functools.partial(
    pl.kernel,
    out_type=jax.ShapeDtypeStruct((_NC, 2, _NP), jnp.float32),
    mesh=_mesh,
    compiler_params=pltpu.CompilerParams(use_tc_tiling_on_sc=False),
    scratch_types=[
        pltpu.VMEM((_NB, _B), jnp.int32),
        pltpu.VMEM((_NB, _B), jnp.int32),
        pltpu.VMEM((128,), jnp.float32),
        pltpu.VMEM((_TS,), jnp.float32),
        pltpu.VMEM_SHARED((_NP,), jnp.float32),
        pltpu.VMEM_SHARED((_NP,), jnp.float32),
        pltpu.SemaphoreType.DMA,
    ],
)
def _deg_kernel(src_hbm, dst_hbm, deg_hbm, src_v, dst_v, ones_v, zero_v,
                dego_sh, degi_sh, sem):
    c = lax.axis_index("c")
    s = lax.axis_index("s")
    w = c * _NS + s
    pltpu.sync_copy(src_hbm.at[w], src_v)
    pltpu.sync_copy(dst_hbm.at[w], dst_v)
    _fill_1d(ones_v, 128, 1.0)
    _fill_1d(zero_v, _TS, 0.0)
    pltpu.sync_copy(zero_v, dego_sh.at[pl.ds(s * _TS, _TS)])
    pltpu.sync_copy(zero_v, degi_sh.at[pl.ds(s * _TS, _TS)])
    plsc.subcore_barrier()

    def body(q, carry):
        for u in range(4):
            j = q * 4 + u
            pltpu.async_copy(ones_v.at[pl.ds(0, _B)], dego_sh.at[src_v.at[j]],
                             sem, add=True)
            pltpu.async_copy(ones_v.at[pl.ds(0, _B)], degi_sh.at[dst_v.at[j]],
                             sem, add=True)
        for u in range(8):
            pltpu.make_async_copy(ones_v.at[pl.ds(0, _B)],
                                  dego_sh.at[src_v.at[0]], sem).wait()
        return carry
    lax.fori_loop(0, _NB // 4, body, 0)
    plsc.subcore_barrier()
    sl = pl.ds(s * _TS, _TS)
    pltpu.sync_copy(dego_sh.at[sl], deg_hbm.at[c, 0, sl])
    pltpu.sync_copy(degi_sh.at[sl], deg_hbm.at[c, 1, sl])


def _prep_body(x_ref, deg_ref, xs_ref, norm_ref, nd1_ref):
    deg = deg_ref[0] + deg_ref[1]
    norm = lax.rsqrt(jnp.maximum(deg, 1.0))
    norm_ref[...] = norm
    nd1_ref[...] = norm[1]
    ns_col = jnp.reshape(norm[0, :_N], (_N, 1))
    x = x_ref[...]
    mask = (jnp.sum(x, axis=1, keepdims=True) != 0.0).astype(jnp.float32)
    xs_ref[...] = x * (mask * ns_col)


_RING = 4


@functools.partial(
    pl.kernel,
    out_type=(
        jax.ShapeDtypeStruct((_NP, _D), jnp.float32),
        jax.ShapeDtypeStruct((_NC, _NP), jnp.float32),
    ),
    mesh=_mesh,
    compiler_params=pltpu.CompilerParams(use_tc_tiling_on_sc=False),
    scratch_types=[
        pltpu.VMEM((_NB2, _B), jnp.int32),
        pltpu.VMEM((_NB, _B), jnp.int32),
        pltpu.VMEM((_NB2, _B), jnp.int32),
        pltpu.VMEM((_B, _DH), jnp.float32),
        pltpu.VMEM((_B, _DH), jnp.float32),
        pltpu.VMEM((_B, _DH), jnp.float32),
        pltpu.VMEM((_B, _DH), jnp.float32),
        pltpu.VMEM((_B,), jnp.float32),
        pltpu.VMEM((_B,), jnp.float32),
        pltpu.VMEM((64, _DH), jnp.float32),
        pltpu.VMEM((_TS,), jnp.float32),
        pltpu.VMEM_SHARED((_NP, _DH), jnp.float32),
        pltpu.VMEM_SHARED((_NP,), jnp.float32),
        pltpu.SemaphoreType.DMA,
        pltpu.SemaphoreType.DMA,
        pltpu.SemaphoreType.DMA,
        pltpu.SemaphoreType.DMA,
        pltpu.SemaphoreType.DMA,
        pltpu.SemaphoreType.DMA,
    ],
)
def _edge_kernel(xsr_hbm, nd_hbm, srcx_hbm, srcp_hbm, dst_hbm, agg_hbm, t_hbm,
                 srcx_v, srcp_v, dst_v, rows0, rows1, rows2, rows3,
                 nv0, nv1, zero_v, zt_v, agg_sh, t_sh,
                 semr0, semr1, semr2, semr3, semn0, semn1):
    c = lax.axis_index("c")
    s = lax.axis_index("s")
    rows = (rows0, rows1, rows2, rows3)
    semr = (semr0, semr1, semr2, semr3)
    nvals = (nv0, nv1)
    semn = (semn0, semn1)
    pltpu.sync_copy(srcx_hbm.at[c, s], srcx_v)
    pltpu.sync_copy(srcp_hbm.at[s, pl.ds(c * _NB, _NB)], srcp_v)
    pltpu.sync_copy(dst_hbm.at[s], dst_v)

    for u in range(_RING):
        pltpu.async_copy(xsr_hbm.at[srcx_v.at[u]], rows[u], semr[u])
    for v in range(2):
        pltpu.async_copy(nd_hbm.at[dst_v.at[c * _NB + v]], nvals[v], semn[v])

    def zf(i, carry):
        for k in range(_DH // 16):
            zero_v[i, pl.ds(k * 16, 16)] = jnp.zeros((16,), jnp.float32)
        return carry
    lax.fori_loop(0, 64, zf, 0)
    _fill_1d(zt_v, _TS, 0.0)
    for q in range(_TS // 64):
        pltpu.sync_copy(zero_v, agg_sh.at[pl.ds(s * _TS + q * 64, 64), :])
    pltpu.sync_copy(zt_v, t_sh.at[pl.ds(s * _TS, _TS)])
    plsc.subcore_barrier()

    def body(q, carry):
        for u in range(_RING):
            j = q * _RING + u
            jp = jnp.minimum(j + _RING, _NB2 - 1)
            pltpu.make_async_copy(xsr_hbm.at[srcx_v.at[j]], rows[u],
                                  semr[u]).wait()
            pltpu.sync_copy(rows[u], agg_sh.at[dst_v.at[j]], add=True)
            pltpu.async_copy(xsr_hbm.at[srcx_v.at[jp]], rows[u], semr[u])
            if u % 2 == 1:
                v = u // 2
                jj = q * 2 + v
                jjp = jnp.minimum(c * _NB + jj + 2, _NB2 - 1)
                pltpu.make_async_copy(nd_hbm.at[dst_v.at[c * _NB + jj]],
                                      nvals[v], semn[v]).wait()
                pltpu.sync_copy(nvals[v], t_sh.at[srcp_v.at[jj]], add=True)
                pltpu.async_copy(nd_hbm.at[dst_v.at[jjp]], nvals[v], semn[v])
        return carry
    lax.fori_loop(0, _NB2 // _RING, body, 0)

    for u in range(_RING):
        pltpu.make_async_copy(xsr_hbm.at[srcx_v.at[0]], rows[u],
                              semr[u]).wait()
    for v in range(2):
        pltpu.make_async_copy(nd_hbm.at[dst_v.at[0]], nvals[v],
                              semn[v]).wait()
    plsc.subcore_barrier()
    sl = pl.ds(s * _TS, _TS)
    pltpu.sync_copy(agg_sh.at[sl, :], agg_hbm.at[sl, pl.ds(c * _DH, _DH)])
    pltpu.sync_copy(t_sh.at[sl], t_hbm.at[c, sl])


_RB = 1280


def _final_body(agg_ref, t_ref, norm_ref, w0_ref, b0_ref, w1_ref,
                b1_ref, out_ref, acc_ref):
    i = pl.program_id(0)

    @pl.when(i == 0)
    def _init():
        acc_ref[...] = jnp.zeros_like(acc_ref)

    a = agg_ref[...]
    nd_col = jnp.reshape(norm_ref[1], (_RB, 1))
    z = jnp.dot(a * nd_col, w0_ref[...], preferred_element_type=jnp.float32)
    h = jnp.maximum(z + b0_ref[...], 0.0)
    cvec = norm_ref[0] * (t_ref[0] + t_ref[1])
    acc_ref[...] += jnp.sum(h * jnp.reshape(cvec, (_RB, 1)), axis=0,
                            keepdims=True)

    @pl.when(i == pl.num_programs(0) - 1)
    def _fin():
        v = acc_ref[...] * (1.0 / _N)
        out_ref[...] = (
            jnp.dot(v, w1_ref[...], preferred_element_type=jnp.float32)
            + b1_ref[...]
        )


def kernel(x, edge_index, W0, b0, W1, b1):
    src = edge_index[0]
    dst = edge_index[1]
    src3 = src.reshape(_NW, _NB, _B)
    dst3 = dst.reshape(_NW, _NB, _B)

    deg = _deg_kernel(src3, dst3)

    xs, norm, nd1 = pl.pallas_call(
        _prep_body,
        out_shape=(
            jax.ShapeDtypeStruct((_N, _D), jnp.float32),
            jax.ShapeDtypeStruct((2, _NP), jnp.float32),
            jax.ShapeDtypeStruct((_NP,), jnp.float32),
        ),
    )(x, deg)

    srcx = (src[None, :] * 2
            + jnp.arange(_NC, dtype=src.dtype)[:, None]).reshape(
                _NC, _NS, _NB2, _B)
    srcp = src.reshape(_NS, _NB2, _B)
    dst2 = dst.reshape(_NS, _NB2, _B)
    agg, t = _edge_kernel(xs.reshape(2 * _N, _DH), nd1, srcx, srcp, dst2)

    out = pl.pallas_call(
        _final_body,
        grid=(_NP // _RB,),
        in_specs=[
            pl.BlockSpec((_RB, _D), lambda i: (i, 0)),
            pl.BlockSpec((_NC, _RB), lambda i: (0, i)),
            pl.BlockSpec((2, _RB), lambda i: (0, i)),
            pl.BlockSpec((_D, _D), lambda i: (0, 0)),
            pl.BlockSpec((1, _D), lambda i: (0, 0)),
            pl.BlockSpec((_D, _D), lambda i: (0, 0)),
            pl.BlockSpec((1, _D), lambda i: (0, 0)),
        ],
        out_specs=pl.BlockSpec((1, _D), lambda i: (0, 0)),
        out_shape=jax.ShapeDtypeStruct((1, _D), jnp.float32),
        scratch_shapes=[pltpu.VMEM((1, _D), jnp.float32)],
    )(agg, t, norm, W0, b0.reshape(1, _D), W1, b1.reshape(1, _D))
    return out
